# baseline (speedup 1.0000x reference)
import jax
import jax.numpy as jnp
from jax import lax
from jax.experimental import pallas as pl
from jax.experimental.pallas import tpu as pltpu
from jax.experimental.pallas import tpu_sc as plsc

NC = 2
NS = 16
NW = NC * NS

BATCH = 4096
SEQ = 200
D = 32
PD = 64
CB = BATCH // NW
RPT = CB + 1


def _body(table_hbm, xt_hbm, out_hbm, idx_v, r0, r1, t0, t1,
          gsem0, gsem1, osem0, osem1):
    cid = lax.axis_index("c")
    sid = lax.axis_index("s")
    wid = sid * NC + cid

    rbuf = (r0, r1)
    tbuf = (t0, t1)
    gsem = (gsem0, gsem1)
    osem = (osem0, osem1)

    pltpu.sync_copy(xt_hbm.at[:, pl.ds(wid * CB, CB)], idx_v)

    iota = lax.iota(jnp.int32, 16)
    fvec = [iota + 16 * h for h in range(2)]

    def fire_gather(s, b):
        return pltpu.async_copy(table_hbm.at[idx_v.at[s]], rbuf[b], gsem[b])

    def wait_gather(s, b):
        pltpu.make_async_copy(table_hbm.at[idx_v.at[s]], rbuf[b],
                              gsem[b]).wait()

    def transpose(b):
        for r in range(CB):
            rv = jnp.full((16,), r, jnp.int32)
            for h in range(2):
                v = rbuf[b][r, pl.ds(16 * h, 16)]
                plsc.store_scatter(tbuf[b], [fvec[h], rv], v)

    def fire_out(s, b):
        for dt in range(D // 8):
            pltpu.async_copy(tbuf[b].at[pl.ds(dt * 8, 8), pl.ds(0, CB)],
                             out_hbm.at[s, dt, wid], osem[b])

    def wait_out(s, b):
        for dt in range(D // 8):
            pltpu.make_async_copy(tbuf[b].at[pl.ds(dt * 8, 8), pl.ds(0, CB)],
                                  out_hbm.at[s, dt, wid], osem[b]).wait()

    def step(s, b, first):
        wait_gather(s, b)
        if not first:
            wait_out(s, b)
        transpose(b)
        fire_out(s, b)

    fire_gather(0, 0)
    fire_gather(1, 1)
    step(0, 0, True)
    fire_gather(2, 0)
    step(1, 1, True)

    def pair(t, carry):
        for b in range(2):
            s = 2 * t + b
            fire_gather(s + 1, 1 - b)
            step(s, b, False)
        return carry

    lax.fori_loop(1, SEQ // 2 - 1, pair, 0)

    fire_gather(SEQ - 1, 1)
    step(SEQ - 2, 0, False)
    step(SEQ - 1, 1, False)
    wait_out(SEQ - 2, 0)
    wait_out(SEQ - 1, 1)


@jax.jit
def _embed(xt, tpad):
    mesh = plsc.VectorSubcoreMesh(
        core_axis_name="c", subcore_axis_name="s",
        num_cores=NC, num_subcores=NS,
    )
    f = pl.kernel(
        _body,
        out_type=jax.ShapeDtypeStruct((SEQ, D // 8, NW, 8, CB), jnp.float32),
        mesh=mesh,
        scratch_types=[
            pltpu.VMEM((SEQ, CB), jnp.int32),
            pltpu.VMEM((CB, D), jnp.float32),
            pltpu.VMEM((CB, D), jnp.float32),
            pltpu.VMEM((D, RPT), jnp.float32),
            pltpu.VMEM((D, RPT), jnp.float32),
            pltpu.SemaphoreType.DMA,
            pltpu.SemaphoreType.DMA,
            pltpu.SemaphoreType.DMA,
            pltpu.SemaphoreType.DMA,
        ],
        compiler_params=pltpu.CompilerParams(
            use_tc_tiling_on_sc=False, needs_layout_passes=False),
    )
    return f(tpad, xt)


def kernel(x, table):
    xt = (x.astype(jnp.int32) * (PD // D)).T
    tpad = jnp.pad(table, ((0, 0), (0, PD - D))).reshape(-1, D)
    out5d = _embed(xt, tpad)
    return out5d.transpose(2, 4, 0, 1, 3).reshape(BATCH, SEQ, D)


# speedup vs baseline: 1.5379x; 1.5379x over previous
import jax
import jax.numpy as jnp
from jax import lax
from jax.experimental import pallas as pl
from jax.experimental.pallas import tpu as pltpu
from jax.experimental.pallas import tpu_sc as plsc

NC = 2
NS = 16
NW = NC * NS

BATCH = 4096
SEQ = 200
D = 32
PD = 128
CB = BATCH // NW
RPT = CB + 1


def _body(table_hbm, xt_hbm, out_hbm, idx_v, r0, r1, t0, t1,
          gsem0, gsem1, osem0, osem1):
    cid = lax.axis_index("c")
    sid = lax.axis_index("s")
    wid = sid * NC + cid

    rbuf = (r0, r1)
    tbuf = (t0, t1)
    gsem = (gsem0, gsem1)
    osem = (osem0, osem1)

    pltpu.sync_copy(xt_hbm.at[:, pl.ds(wid * CB, CB)], idx_v)

    iota = lax.iota(jnp.int32, 16)
    fvec = [iota + 16 * h for h in range(2)]

    def fire_gather(s, b):
        return pltpu.async_copy(table_hbm.at[idx_v.at[s]], rbuf[b], gsem[b])

    def wait_gather(s, b):
        pltpu.make_async_copy(table_hbm.at[idx_v.at[s]], rbuf[b],
                              gsem[b]).wait()

    def transpose(b):
        for r in range(CB):
            rv = jnp.full((16,), r, jnp.int32)
            for h in range(2):
                v = rbuf[b][r, pl.ds(16 * h, 16)]
                plsc.store_scatter(tbuf[b], [fvec[h], rv], v)

    def fire_out(s, b):
        for dt in range(D // 8):
            pltpu.async_copy(tbuf[b].at[pl.ds(dt * 8, 8), pl.ds(0, CB)],
                             out_hbm.at[s, dt, wid], osem[b])

    def wait_out(s, b):
        for dt in range(D // 8):
            pltpu.make_async_copy(tbuf[b].at[pl.ds(dt * 8, 8), pl.ds(0, CB)],
                                  out_hbm.at[s, dt, wid], osem[b]).wait()

    def step(s, b, first):
        wait_gather(s, b)
        if not first:
            wait_out(s, b)
        transpose(b)
        fire_out(s, b)

    fire_gather(0, 0)
    fire_gather(1, 1)
    step(0, 0, True)
    fire_gather(2, 0)
    step(1, 1, True)

    def pair(t, carry):
        for b in range(2):
            s = 2 * t + b
            fire_gather(s + 1, 1 - b)
            step(s, b, False)
        return carry

    lax.fori_loop(1, SEQ // 2 - 1, pair, 0)

    fire_gather(SEQ - 1, 1)
    step(SEQ - 2, 0, False)
    step(SEQ - 1, 1, False)
    wait_out(SEQ - 2, 0)
    wait_out(SEQ - 1, 1)


@jax.jit
def _embed(xt, tpad):
    mesh = plsc.VectorSubcoreMesh(
        core_axis_name="c", subcore_axis_name="s",
        num_cores=NC, num_subcores=NS,
    )
    f = pl.kernel(
        _body,
        out_type=jax.ShapeDtypeStruct((SEQ, D // 8, NW, 8, CB), jnp.float32),
        mesh=mesh,
        scratch_types=[
            pltpu.VMEM((SEQ, CB), jnp.int32),
            pltpu.VMEM((CB, D), jnp.float32),
            pltpu.VMEM((CB, D), jnp.float32),
            pltpu.VMEM((D, RPT), jnp.float32),
            pltpu.VMEM((D, RPT), jnp.float32),
            pltpu.SemaphoreType.DMA,
            pltpu.SemaphoreType.DMA,
            pltpu.SemaphoreType.DMA,
            pltpu.SemaphoreType.DMA,
        ],
        compiler_params=pltpu.CompilerParams(
            use_tc_tiling_on_sc=False, needs_layout_passes=False),
    )
    return f(tpad, xt)


def kernel(x, table):
    xt = (x.astype(jnp.int32) * (PD // D)).T
    tpad = jnp.pad(table, ((0, 0), (0, PD - D))).reshape(-1, D)
    out5d = _embed(xt, tpad)
    return out5d.transpose(2, 4, 0, 1, 3).reshape(BATCH, SEQ, D)
